# baseline (speedup 1.0000x reference)
import functools

import jax
import jax.numpy as jnp
from jax.experimental import pallas as pl
from jax.experimental.pallas import tpu as pltpu


def _fused_kernel(x_ref, w1_ref, b1_ref, w2_ref, b2_ref,
                  wp1_ref, bp1_ref, wp2_ref, bp2_ref, bm_ref, bmm_ref,
                  o_ref, *, NT):
    i = pl.program_id(0)
    x = x_ref[...]

    @pl.when(i == 0)
    def _init():
        o_ref[...] = b2_ref[...] + jnp.dot(
            bm_ref[...], bp2_ref[...], preferred_element_type=jnp.float32)

    @pl.when(i < NT)
    def _target_chunk():
        h = jnp.dot(x, w1_ref[...], preferred_element_type=jnp.float32)
        h = jnp.maximum(h + b1_ref[...], 0.0)
        o_ref[...] += jnp.dot(h, w2_ref[...],
                              preferred_element_type=jnp.float32)

    @pl.when(i >= NT)
    def _expert_chunk():
        he = jnp.dot(x, wp1_ref[0], preferred_element_type=jnp.float32)
        he = jnp.maximum(he + bp1_ref[0], 0.0) * bmm_ref[0]
        o_ref[...] += jnp.dot(he, wp2_ref[0],
                              preferred_element_type=jnp.float32)


def kernel(x, in_bitmap, W1, b1, W2, b2, Wp1, bp1, Wp2, bp2):
    N, D = x.shape
    H = W1.shape[1]
    E, _, PH = Wp1.shape
    C = W2.shape[1]
    NT = H // PH

    bm = in_bitmap.astype(jnp.float32)
    bmm = bm.T.reshape(E, N, 1)

    grid = (NT + E,)
    t = lambda i: jnp.minimum(i, NT - 1)
    e = lambda i: jnp.maximum(i - NT, 0)
    out = pl.pallas_call(
        functools.partial(_fused_kernel, NT=NT),
        grid=grid,
        in_specs=[
            pl.BlockSpec((N, D), lambda i: (0, 0)),
            pl.BlockSpec((D, PH), lambda i: (0, t(i))),
            pl.BlockSpec((1, PH), lambda i: (0, t(i))),
            pl.BlockSpec((PH, C), lambda i: (t(i), 0)),
            pl.BlockSpec((1, C), lambda i: (0, 0)),
            pl.BlockSpec((1, D, PH), lambda i: (e(i), 0, 0)),
            pl.BlockSpec((1, 1, PH), lambda i: (e(i), 0, 0)),
            pl.BlockSpec((1, PH, C), lambda i: (e(i), 0, 0)),
            pl.BlockSpec((E, C), lambda i: (0, 0)),
            pl.BlockSpec((N, E), lambda i: (0, 0)),
            pl.BlockSpec((1, N, 1), lambda i: (e(i), 0, 0)),
        ],
        out_specs=pl.BlockSpec((N, C), lambda i: (0, 0)),
        out_shape=jax.ShapeDtypeStruct((N, C), jnp.float32),
        compiler_params=pltpu.CompilerParams(
            dimension_semantics=("arbitrary",),
        ),
    )(x, W1, b1.reshape(1, H), W2, b2.reshape(1, C),
      Wp1, bp1.reshape(E, 1, PH), Wp2, bp2, bm, bmm)
    return out


# speedup vs baseline: 1.6399x; 1.6399x over previous
import functools

import jax
import jax.numpy as jnp
from jax.experimental import pallas as pl
from jax.experimental.pallas import tpu as pltpu


def _fused_kernel(x_ref, bm_ref, w1_ref, b1_ref, w2_ref, b2_ref,
                  wp1_ref, bp1_ref, wp2_ref, bp2_ref, o_ref, *, E, PH):
    x = x_ref[...]
    bm = bm_ref[...]
    H = w1_ref.shape[1]
    o = b2_ref[...] + jnp.dot(bm, bp2_ref[...],
                              preferred_element_type=jnp.float32)
    for c in range(H // PH):
        sl = pl.ds(c * PH, PH)
        h = jnp.dot(x, w1_ref[:, sl], preferred_element_type=jnp.float32)
        h = jnp.maximum(h + b1_ref[:, sl], 0.0)
        o = o + jnp.dot(h, w2_ref[sl, :], preferred_element_type=jnp.float32)
    for e in range(E):
        he = jnp.dot(x, wp1_ref[e], preferred_element_type=jnp.float32)
        he = jnp.maximum(he + bp1_ref[e], 0.0) * bm[:, e][:, None]
        o = o + jnp.dot(he, wp2_ref[e], preferred_element_type=jnp.float32)
    o_ref[...] = o


def kernel(x, in_bitmap, W1, b1, W2, b2, Wp1, bp1, Wp2, bp2):
    N, D = x.shape
    H = W1.shape[1]
    E, _, PH = Wp1.shape
    C = W2.shape[1]

    bm = in_bitmap.astype(jnp.float32)

    BN = 4096
    grid = (N // BN,)
    out = pl.pallas_call(
        functools.partial(_fused_kernel, E=E, PH=PH),
        grid=grid,
        in_specs=[
            pl.BlockSpec((BN, D), lambda i: (i, 0)),
            pl.BlockSpec((BN, E), lambda i: (i, 0)),
            pl.BlockSpec((D, H), lambda i: (0, 0)),
            pl.BlockSpec((1, H), lambda i: (0, 0)),
            pl.BlockSpec((H, C), lambda i: (0, 0)),
            pl.BlockSpec((1, C), lambda i: (0, 0)),
            pl.BlockSpec((E, D, PH), lambda i: (0, 0, 0)),
            pl.BlockSpec((E, PH), lambda i: (0, 0)),
            pl.BlockSpec((E, PH, C), lambda i: (0, 0, 0)),
            pl.BlockSpec((E, C), lambda i: (0, 0)),
        ],
        out_specs=pl.BlockSpec((BN, C), lambda i: (i, 0)),
        out_shape=jax.ShapeDtypeStruct((N, C), jnp.float32),
        compiler_params=pltpu.CompilerParams(
            dimension_semantics=("arbitrary",),
        ),
    )(x, bm, W1, b1.reshape(1, H), W2, b2.reshape(1, C), Wp1, bp1, Wp2, bp2)
    return out
